# baseline (speedup 1.0000x reference)
import functools

import jax
import jax.numpy as jnp
from jax import lax
from jax.experimental import pallas as pl
from jax.experimental.pallas import tpu as pltpu
from jax.experimental.pallas import tpu_sc as plsc

_EMB = 32
_N = 16384
_NC = 2
_NS = 16
_NW = _NC * _NS
_BPW = _N // _NW
_CH = 128
_NCHUNK = _BPW // _CH

_mesh = plsc.VectorSubcoreMesh(core_axis_name="c", subcore_axis_name="s")


@functools.partial(
    pl.kernel,
    mesh=_mesh,
    compiler_params=pltpu.CompilerParams(
        needs_layout_passes=False, use_tc_tiling_on_sc=False
    ),
    out_type=[
        jax.ShapeDtypeStruct((_N,), jnp.float32),
        jax.ShapeDtypeStruct((_N, _EMB), jnp.float32),
        jax.ShapeDtypeStruct((_N, _EMB), jnp.float32),
        jax.ShapeDtypeStruct((_N, _EMB), jnp.float32),
        jax.ShapeDtypeStruct((_N, _EMB), jnp.float32),
    ],
    scratch_types=[
        pltpu.VMEM((_NCHUNK, _CH), jnp.int32),
        pltpu.VMEM((_NCHUNK, _CH), jnp.int32),
        pltpu.VMEM((_NCHUNK, _CH), jnp.int32),
        pltpu.VMEM((_NCHUNK, _CH), jnp.int32),
        pltpu.VMEM((_BPW, _EMB), jnp.float32),
        pltpu.VMEM((_BPW, _EMB), jnp.float32),
        pltpu.VMEM((_BPW, _EMB), jnp.float32),
        pltpu.VMEM((_BPW, _EMB), jnp.float32),
        pltpu.VMEM((_BPW,), jnp.float32),
        pltpu.SemaphoreType.DMA,
        pltpu.SemaphoreType.DMA,
    ],
)
def _sc_kernel(h_hbm, t_hbm, r_hbm, m_hbm, ent_hbm, rel_hbm, tem_hbm,
               scores_out, he_out, te_out, re_out, me_out,
               hidx, tidx, ridx, midx, hrows, trows, rrows, mrows,
               sc_v, gsem, osem):
    wid = lax.axis_index("s") * _NC + lax.axis_index("c")
    base = wid * _BPW

    pltpu.sync_copy(h_hbm.at[pl.ds(wid * _NCHUNK, _NCHUNK)], hidx)
    pltpu.sync_copy(t_hbm.at[pl.ds(wid * _NCHUNK, _NCHUNK)], tidx)
    pltpu.sync_copy(r_hbm.at[pl.ds(wid * _NCHUNK, _NCHUNK)], ridx)
    pltpu.sync_copy(m_hbm.at[pl.ds(wid * _NCHUNK, _NCHUNK)], midx)

    copies = []
    for j in range(_NCHUNK):
        dst = pl.ds(j * _CH, _CH)
        copies.append(pltpu.async_copy(ent_hbm.at[hidx.at[j]], hrows.at[dst], gsem))
        copies.append(pltpu.async_copy(ent_hbm.at[tidx.at[j]], trows.at[dst], gsem))
        copies.append(pltpu.async_copy(rel_hbm.at[ridx.at[j]], rrows.at[dst], gsem))
        copies.append(pltpu.async_copy(tem_hbm.at[midx.at[j]], mrows.at[dst], gsem))
    for c in copies:
        c.wait()

    out_copies = [
        pltpu.async_copy(hrows, he_out.at[pl.ds(base, _BPW)], osem),
        pltpu.async_copy(trows, te_out.at[pl.ds(base, _BPW)], osem),
        pltpu.async_copy(rrows, re_out.at[pl.ds(base, _BPW)], osem),
        pltpu.async_copy(mrows, me_out.at[pl.ds(base, _BPW)], osem),
    ]

    lane = lax.iota(jnp.int32, 16)

    def body(g, carry):
        rowids = g * 16 + lane
        acc0 = jnp.zeros((16,), jnp.float32)
        acc1 = jnp.zeros((16,), jnp.float32)
        for d in range(_EMB):
            dd = jnp.full((16,), d, jnp.int32)
            hv = plsc.load_gather(hrows, [rowids, dd])
            rv = plsc.load_gather(rrows, [rowids, dd])
            mv = plsc.load_gather(mrows, [rowids, dd])
            tv = plsc.load_gather(trows, [rowids, dd])
            term = jnp.abs(hv + rv + mv - tv)
            if d % 2 == 0:
                acc0 = acc0 + term
            else:
                acc1 = acc1 + term
        sc_v[pl.ds(pl.multiple_of(g * 16, 16), 16)] = -(acc0 + acc1)
        return carry

    lax.fori_loop(0, _BPW // 16, body, 0)
    pltpu.sync_copy(sc_v, scores_out.at[pl.ds(base, _BPW)])
    for c in out_copies:
        c.wait()


def kernel(samples, ent_w, rel_w, tem_w):
    h = samples[:, 0].reshape(_N // _CH, _CH)
    r = samples[:, 1].reshape(_N // _CH, _CH)
    t = samples[:, 2].reshape(_N // _CH, _CH)
    tem = samples[:, 3].reshape(_N // _CH, _CH)
    scores, h_e, t_e, r_e, tem_e = _sc_kernel(h, t, r, tem, ent_w, rel_w, tem_w)
    return (scores, h_e, t_e, r_e, tem_e)


# speedup vs baseline: 1.0298x; 1.0298x over previous
import functools

import jax
import jax.numpy as jnp
from jax import lax
from jax.experimental import pallas as pl
from jax.experimental.pallas import tpu as pltpu
from jax.experimental.pallas import tpu_sc as plsc

_EMB = 32
_N = 16384
_NC = 2
_NS = 16
_NW = _NC * _NS
_BPW = _N // _NW
_CH = 128
_NCHUNK = _BPW // _CH
_SUB = 64
_NSUB = _BPW // _SUB

_mesh = plsc.VectorSubcoreMesh(core_axis_name="c", subcore_axis_name="s")


@functools.partial(
    pl.kernel,
    mesh=_mesh,
    compiler_params=pltpu.CompilerParams(needs_layout_passes=False),
    out_type=[
        jax.ShapeDtypeStruct((_N,), jnp.float32),
        jax.ShapeDtypeStruct((_EMB, _N), jnp.float32),
        jax.ShapeDtypeStruct((_EMB, _N), jnp.float32),
        jax.ShapeDtypeStruct((_EMB, _N), jnp.float32),
        jax.ShapeDtypeStruct((_EMB, _N), jnp.float32),
    ],
    scratch_types=[
        pltpu.VMEM((_NSUB, _SUB), jnp.int32),
        pltpu.VMEM((_NSUB, _SUB), jnp.int32),
        pltpu.VMEM((_NSUB, _SUB), jnp.int32),
        pltpu.VMEM((_NSUB, _SUB), jnp.int32),
        pltpu.VMEM((_NSUB, _SUB), jnp.int32),
        pltpu.VMEM((_NSUB, _SUB), jnp.int32),
        pltpu.VMEM((2, _SUB, 128), jnp.float32),
        pltpu.VMEM((2, _SUB, 128), jnp.float32),
        pltpu.VMEM((2, _SUB, 128), jnp.float32),
        pltpu.VMEM((2, _SUB, 128), jnp.float32),
        pltpu.VMEM((_EMB, _CH), jnp.float32),
        pltpu.VMEM((_EMB, _CH), jnp.float32),
        pltpu.VMEM((_EMB, _CH), jnp.float32),
        pltpu.VMEM((_EMB, _CH), jnp.float32),
        pltpu.VMEM((_BPW,), jnp.float32),
        pltpu.SemaphoreType.DMA,
        pltpu.SemaphoreType.DMA,
        pltpu.SemaphoreType.DMA,
    ],
)
def _sc_kernel(samples_hbm, ent_hbm, rel_hbm, tem_hbm,
               scores_out, he_out, te_out, re_out, me_out,
               hidx, tidx, ridx, midx, hrow4, trow4,
               hbuf, tbuf, rbuf, mbuf,
               hblk, tblk, rblk, mblk,
               sc_v, gsemA, gsemB, osem):
    wid = lax.axis_index("s") * _NC + lax.axis_index("c")
    base = wid * _BPW

    for k in range(_NSUB):
        sl = pl.ds(base + k * _SUB, _SUB)
        pltpu.sync_copy(samples_hbm.at[0, sl], hidx.at[k])
        pltpu.sync_copy(samples_hbm.at[1, sl], ridx.at[k])
        pltpu.sync_copy(samples_hbm.at[2, sl], tidx.at[k])
        pltpu.sync_copy(samples_hbm.at[3, sl], midx.at[k])

    for k in range(_NSUB):
        for v in range(_SUB // 16):
            o = pl.ds(v * 16, 16)
            hrow4[k, o] = jnp.right_shift(hidx[k, o], 2)
            trow4[k, o] = jnp.right_shift(tidx[k, o], 2)

    lane = lax.iota(jnp.int32, 16)

    def fire(k):
        b = k % 2
        sem = gsemA if b == 0 else gsemB
        return [
            pltpu.async_copy(ent_hbm.at[hrow4.at[k]], hbuf.at[b], sem),
            pltpu.async_copy(ent_hbm.at[trow4.at[k]], tbuf.at[b], sem),
            pltpu.async_copy(rel_hbm.at[ridx.at[k]], rbuf.at[b], sem),
            pltpu.async_copy(tem_hbm.at[midx.at[k]], mbuf.at[b], sem),
        ]

    out_copies = []
    pending = fire(0)
    for k in range(_NSUB):
        b = k % 2
        nxt = fire(k + 1) if k + 1 < _NSUB else []
        for c in pending:
            c.wait()
        pending = nxt
        if b == 0 and k > 0:
            for c in out_copies:
                c.wait()

        def body(g, carry, k=k, b=b):
            o = pl.ds(pl.multiple_of(b * _SUB + g * 16, 16), 16)
            rows16 = g * 16 + lane
            oi = pl.ds(pl.multiple_of(g * 16, 16), 16)
            offh = jnp.left_shift(jnp.bitwise_and(hidx[k, oi], 3), 5)
            offt = jnp.left_shift(jnp.bitwise_and(tidx[k, oi], 3), 5)
            acc0 = jnp.zeros((16,), jnp.float32)
            acc1 = jnp.zeros((16,), jnp.float32)
            for d in range(_EMB):
                dd = jnp.full((16,), d, jnp.int32)
                hv = plsc.load_gather(hbuf.at[b], [rows16, offh + d])
                tv = plsc.load_gather(tbuf.at[b], [rows16, offt + d])
                rv = plsc.load_gather(rbuf.at[b], [rows16, dd])
                mv = plsc.load_gather(mbuf.at[b], [rows16, dd])
                hblk[d, o] = hv
                tblk[d, o] = tv
                rblk[d, o] = rv
                mblk[d, o] = mv
                term = jnp.abs(hv + rv + mv - tv)
                if d % 2 == 0:
                    acc0 = acc0 + term
                else:
                    acc1 = acc1 + term
            sc_v[pl.ds(pl.multiple_of(k * _SUB + g * 16, 16), 16)] = -(acc0 + acc1)
            return carry

        lax.fori_loop(0, _SUB // 16, body, 0)

        if b == 1:
            osl = pl.ds(base + (k // 2) * _CH, _CH)
            out_copies = [
                pltpu.async_copy(hblk, he_out.at[:, osl], osem),
                pltpu.async_copy(tblk, te_out.at[:, osl], osem),
                pltpu.async_copy(rblk, re_out.at[:, osl], osem),
                pltpu.async_copy(mblk, me_out.at[:, osl], osem),
            ]

    pltpu.sync_copy(sc_v, scores_out.at[pl.ds(base, _BPW)])
    for c in out_copies:
        c.wait()


def kernel(samples, ent_w, rel_w, tem_w):
    samples_t = samples.T
    ent4 = ent_w.reshape(250000, 128)
    rel_p = jnp.pad(rel_w, ((0, 0), (0, 96)))
    tem_p = jnp.pad(tem_w, ((0, 0), (0, 96)))
    scores, h_eT, t_eT, r_eT, tem_eT = _sc_kernel(samples_t, ent4, rel_p, tem_p)
    return (scores, h_eT.T, t_eT.T, r_eT.T, tem_eT.T)
